# Initial kernel scaffold; baseline (speedup 1.0000x reference)
import numpy as np
import jax
import jax.numpy as jnp
from jax.experimental import pallas as pl

_ALPHA = 4


def _copy_body(src_ref, dst_ref):
    dst_ref[...] = src_ref[...]


def kernel(frames):
    temporal_axis = 1 if frames.ndim == 4 else 2
    T = frames.shape[temporal_axis]
    S = T // _ALPHA
    idx = tuple(int(v) for v in np.linspace(0.0, T - 1, S))

    if frames.ndim == 4:
        C, _, H, W = frames.shape
        lead = C
    else:
        B, C, _, H, W = frames.shape
        lead = B * C

    x = frames.reshape(lead, T, H * W)
    idx_arr = jnp.asarray(idx, dtype=jnp.int32)

    slow = pl.pallas_call(
        _copy_body,
        grid=(lead, S),
        in_specs=[
            pl.BlockSpec((1, 1, H * W), lambda i, t: (i, idx_arr[t], 0)),
        ],
        out_specs=pl.BlockSpec((1, 1, H * W), lambda i, t: (i, t, 0)),
        out_shape=jax.ShapeDtypeStruct((lead, S, H * W), frames.dtype),
    )(x)

    if frames.ndim == 4:
        slow = slow.reshape(C, S, H, W)
    else:
        slow = slow.reshape(B, C, S, H, W)
    return (slow, frames)


# speedup vs baseline: 1.2039x; 1.2039x over previous
import numpy as np
import jax
import jax.numpy as jnp
from jax.experimental import pallas as pl
from jax.experimental.pallas import tpu as pltpu

_ALPHA = 4


def _copy_body(idx_ref, src_ref, dst_ref):
    del idx_ref
    dst_ref[...] = src_ref[...]


def kernel(frames):
    temporal_axis = 1 if frames.ndim == 4 else 2
    T = frames.shape[temporal_axis]
    S = T // _ALPHA
    idx = tuple(int(v) for v in np.linspace(0.0, T - 1, S))

    if frames.ndim == 4:
        C, _, H, W = frames.shape
        lead = C
    else:
        B, C, _, H, W = frames.shape
        lead = B * C

    hw = H * W
    lanes = 128
    rows = hw // lanes
    x = frames.reshape(lead, T, rows, lanes)
    idx_arr = jnp.asarray(idx, dtype=jnp.int32)

    slow = pl.pallas_call(
        _copy_body,
        grid_spec=pltpu.PrefetchScalarGridSpec(
            num_scalar_prefetch=1,
            grid=(lead, S),
            in_specs=[
                pl.BlockSpec((1, 1, rows, lanes),
                             lambda i, t, idx_ref: (i, idx_ref[t], 0, 0)),
            ],
            out_specs=pl.BlockSpec((1, 1, rows, lanes),
                                   lambda i, t, idx_ref: (i, t, 0, 0)),
        ),
        out_shape=jax.ShapeDtypeStruct((lead, S, rows, lanes), frames.dtype),
    )(idx_arr, x)

    if frames.ndim == 4:
        slow = slow.reshape(C, S, H, W)
    else:
        slow = slow.reshape(B, C, S, H, W)
    return (slow, frames)
